# baseline (speedup 1.0000x reference)
import functools

import jax
import jax.numpy as jnp
import numpy as np
from jax.experimental import pallas as pl
from jax.experimental.pallas import tpu as pltpu
from jax.experimental.pallas import tpu_sc as plsc

_N, _E, _D, _H = 10000, 320000, 128, 128
_HH = _H // 2
_BN_EPS = 1e-5

_NC, _NS = 2, 16
_NW = _NC * _NS
_CH = 128
_NB = 4
_EPT = 20480
_EPAD = _NS * _EPT
_NCH = _EPT // _CH
_NG = _NCH // _NB
_NPAD = 10112
_RPT = _NPAD // _NS

_RB = 2000
_GRID = _N // _RB


def _dotT(a, w):
    return jax.lax.dot_general(a, w, (((1,), (1,)), ((), ())),
                               preferred_element_type=jnp.float32)


def _stage_a_body(x_ref, wl_ref, wr_ref, y_ref, r_ref):
    xb = x_ref[...]
    y_ref[...] = _dotT(xb, wl_ref[...])
    r_ref[...] = _dotT(xb, wr_ref[...])


def _stage_a(x, wl, wr):
    blk = lambda: pl.BlockSpec((_RB, _H), lambda i: (i, 0))
    wspec = pl.BlockSpec((_H, _H), lambda i: (0, 0))
    return pl.pallas_call(
        _stage_a_body,
        grid=(_GRID,),
        in_specs=[blk(), wspec, wspec],
        out_specs=[blk(), blk()],
        out_shape=[jax.ShapeDtypeStruct((_N, _H), jnp.float32)] * 2,
    )(x, wl, wr)


def _stage_b_body(agg_ref, r_ref, v_ref, wl_ref, wr_ref, y_ref, o_ref):
    z = agg_ref[...] + r_ref[...] + v_ref[0:1, :]
    h = jnp.maximum(z * v_ref[1:2, :] + v_ref[2:3, :], 0.0)
    y_ref[...] = _dotT(h, wl_ref[...])
    o_ref[...] = _dotT(h, wr_ref[...])


def _stage_b(agg, r, vec, wl, wr):
    blk = lambda: pl.BlockSpec((_RB, _H), lambda i: (i, 0))
    wspec = pl.BlockSpec((_H, _H), lambda i: (0, 0))
    vspec = pl.BlockSpec((8, _H), lambda i: (0, 0))
    return pl.pallas_call(
        _stage_b_body,
        grid=(_GRID,),
        in_specs=[blk(), blk(), vspec, wspec, wspec],
        out_specs=[blk(), blk()],
        out_shape=[jax.ShapeDtypeStruct((_N, _H), jnp.float32)] * 2,
    )(agg, r, vec, wl, wr)


def _stage_c_body(agg_ref, r_ref, v_ref, w0_ref, w1_ref, w2_ref, o_ref):
    z = agg_ref[...] + r_ref[...] + v_ref[0:1, :]
    h = jnp.maximum(z * v_ref[1:2, :] + v_ref[2:3, :], 0.0)
    t = jnp.maximum(_dotT(h, w0_ref[...]) + v_ref[3:4, :], 0.0)
    t = jnp.maximum(_dotT(t, w1_ref[...]) + v_ref[4:5, :], 0.0)
    o_ref[...] = _dotT(t, w2_ref[...]) + v_ref[5:6, :]


def _stage_c(agg, r, vec, w0, w1, w2):
    blk = lambda: pl.BlockSpec((_RB, _H), lambda i: (i, 0))
    wspec = pl.BlockSpec((_H, _H), lambda i: (0, 0))
    vspec = pl.BlockSpec((8, _H), lambda i: (0, 0))
    return pl.pallas_call(
        _stage_c_body,
        grid=(_GRID,),
        in_specs=[blk(), blk(), vspec, wspec, wspec, wspec],
        out_specs=pl.BlockSpec((_RB, _H), lambda i: (i, 0)),
        out_shape=jax.ShapeDtypeStruct((_N, _H), jnp.float32),
    )(agg, r, vec, w0, w1, w2)


def _sc_segsum(tab2, srcp2, dstp, zrows):
    mesh = plsc.VectorSubcoreMesh(core_axis_name="c", subcore_axis_name="s")

    @functools.partial(
        pl.kernel,
        out_type=jax.ShapeDtypeStruct((2 * _NPAD, _HH), jnp.float32),
        mesh=mesh,
        scratch_types=[
            pltpu.VMEM((_NCH, _CH), jnp.int32),
            pltpu.VMEM((_NCH, _CH), jnp.int32),
            [pltpu.VMEM((_CH, _HH), jnp.float32) for _ in range(_NB)],
            pltpu.VMEM_SHARED((_NPAD, _HH), jnp.float32),
            [pltpu.SemaphoreType.DMA for _ in range(_NB)],
            [pltpu.SemaphoreType.DMA for _ in range(_NB)],
            pltpu.SemaphoreType.DMA,
        ],
        compiler_params=pltpu.CompilerParams(use_tc_tiling_on_sc=False),
    )
    def run(tab_hbm, src_hbm, dst_hbm, z_hbm, out_hbm,
            sidx, didx, rows, acc, gsem, ssem, zsem):
        cid = jax.lax.axis_index("c")
        sid = jax.lax.axis_index("s")
        wid = cid * _NS + sid
        row0 = sid * _RPT
        pltpu.async_copy(z_hbm, acc.at[pl.ds(row0, _RPT)], zsem)
        pltpu.sync_copy(src_hbm.at[wid], sidx)
        pltpu.sync_copy(dst_hbm.at[sid], didx)
        pltpu.make_async_copy(z_hbm, acc.at[pl.ds(row0, _RPT)], zsem).wait()
        plsc.subcore_barrier()

        for b in range(_NB):
            pltpu.async_copy(tab_hbm.at[pl.ds(0, _CH)], rows[b], gsem[b])

        def round_(g, carry):
            for b in range(_NB):
                k = g * _NB + b
                pltpu.make_async_copy(tab_hbm.at[pl.ds(0, _CH)],
                                      rows[b], gsem[b]).wait()
                pltpu.async_copy(rows[b], acc.at[didx.at[k]],
                                 ssem[b], add=True)
                pltpu.make_async_copy(rows[b], acc.at[didx.at[k]],
                                      ssem[b]).wait()

                @pl.when(g < _NG - 1)
                def _():
                    pltpu.async_copy(tab_hbm.at[pl.ds(0, _CH)],
                                     rows[b], gsem[b])
            return carry

        jax.lax.fori_loop(0, _NG, round_, 0)
        plsc.subcore_barrier()
        pltpu.sync_copy(acc.at[pl.ds(row0, _RPT)],
                        out_hbm.at[pl.ds(cid * _NPAD + row0, _RPT)])

    return run(tab2, srcp2, dstp, zrows)


def _segsum_full(y, srcp2, dstp, zrows):
    tab2 = jnp.concatenate([y[:, :_HH], y[:, _HH:]], axis=0)
    parts = _sc_segsum(tab2, srcp2, dstp, zrows)
    return jnp.concatenate([parts[:_N], parts[_NPAD:_NPAD + _N]], axis=1)


def kernel(x, edge_index, Wl0, bl0, Wr0, Wl1, bl1, Wr1,
           g0, be0, g1, be1, LW0, Lb0, LW1, Lb1, LW2, Lb2):
    src = edge_index[0]
    dst = edge_index[1]
    pad = _EPAD - _E
    srcp = jnp.concatenate([src, jnp.zeros((pad,), jnp.int32)])
    srcp2 = jnp.concatenate([srcp, srcp + _N]).reshape(_NW, _NCH, _CH)
    dstp = jnp.concatenate([dst, jnp.full((pad,), _N, jnp.int32)])
    dstp = dstp.reshape(_NS, _NCH, _CH)
    zrows = jnp.zeros((_RPT, _HH), jnp.float32)

    c = 1.0 / np.sqrt(1.0 + _BN_EPS)
    zero = jnp.zeros((_H,), jnp.float32)
    vec_b = jnp.stack([bl0, g0 * c, be0, zero, zero, zero, zero, zero])
    vec_c = jnp.stack([bl1, g1 * c, be1, Lb0, Lb1, Lb2, zero, zero])

    y0, r0 = _stage_a(x, Wl0, Wr0)
    agg0 = _segsum_full(y0, srcp2, dstp, zrows)
    y1, r1 = _stage_b(agg0, r0, vec_b, Wl1, Wr1)
    agg1 = _segsum_full(y1, srcp2, dstp, zrows)
    return _stage_c(agg1, r1, vec_c, LW0, LW1, LW2)


# speedup vs baseline: 2.1723x; 2.1723x over previous
import functools

import jax
import jax.numpy as jnp
import numpy as np
from jax.experimental import pallas as pl
from jax.experimental.pallas import tpu as pltpu
from jax.experimental.pallas import tpu_sc as plsc

_N, _E, _D, _H = 10000, 320000, 128, 128
_HH = _H // 2
_BN_EPS = 1e-5

_NC, _NS = 2, 16
_NW = _NC * _NS
_CH = 128
_NB = 4
_EPT = 20480
_EPAD = _NS * _EPT
_NCH = _EPT // _CH
_NG = _NCH // _NB
_NPAD = 10112
_RPT = _NPAD // _NS

_RB = 2000
_GRID = _N // _RB


def _dotT(a, w):
    return jax.lax.dot_general(a, w, (((1,), (1,)), ((), ())),
                               preferred_element_type=jnp.float32)


def _stage_a_body(x_ref, wl_ref, wr_ref, y_ref, r_ref):
    xb = x_ref[...]
    y_ref[...] = _dotT(xb, wl_ref[...])
    r_ref[...] = _dotT(xb, wr_ref[...])


def _stage_a(x, wl, wr):
    blk = lambda: pl.BlockSpec((_RB, _H), lambda i: (i, 0))
    wspec = pl.BlockSpec((_H, _H), lambda i: (0, 0))
    return pl.pallas_call(
        _stage_a_body,
        grid=(_GRID,),
        in_specs=[blk(), wspec, wspec],
        out_specs=[blk(), blk()],
        out_shape=[jax.ShapeDtypeStruct((_N, _H), jnp.float32)] * 2,
    )(x, wl, wr)


def _stage_b_body(agg_ref, r_ref, v_ref, wl_ref, wr_ref, y_ref, o_ref):
    z = agg_ref[...] + r_ref[...] + v_ref[0:1, :]
    h = jnp.maximum(z * v_ref[1:2, :] + v_ref[2:3, :], 0.0)
    y_ref[...] = _dotT(h, wl_ref[...])
    o_ref[...] = _dotT(h, wr_ref[...])


def _stage_b(agg, r, vec, wl, wr):
    blk = lambda: pl.BlockSpec((_RB, _H), lambda i: (i, 0))
    wspec = pl.BlockSpec((_H, _H), lambda i: (0, 0))
    vspec = pl.BlockSpec((8, _H), lambda i: (0, 0))
    return pl.pallas_call(
        _stage_b_body,
        grid=(_GRID,),
        in_specs=[blk(), blk(), vspec, wspec, wspec],
        out_specs=[blk(), blk()],
        out_shape=[jax.ShapeDtypeStruct((_N, _H), jnp.float32)] * 2,
    )(agg, r, vec, wl, wr)


def _stage_c_body(agg_ref, r_ref, v_ref, w0_ref, w1_ref, w2_ref, o_ref):
    z = agg_ref[...] + r_ref[...] + v_ref[0:1, :]
    h = jnp.maximum(z * v_ref[1:2, :] + v_ref[2:3, :], 0.0)
    t = jnp.maximum(_dotT(h, w0_ref[...]) + v_ref[3:4, :], 0.0)
    t = jnp.maximum(_dotT(t, w1_ref[...]) + v_ref[4:5, :], 0.0)
    o_ref[...] = _dotT(t, w2_ref[...]) + v_ref[5:6, :]


def _stage_c(agg, r, vec, w0, w1, w2):
    blk = lambda: pl.BlockSpec((_RB, _H), lambda i: (i, 0))
    wspec = pl.BlockSpec((_H, _H), lambda i: (0, 0))
    vspec = pl.BlockSpec((8, _H), lambda i: (0, 0))
    return pl.pallas_call(
        _stage_c_body,
        grid=(_GRID,),
        in_specs=[blk(), blk(), vspec, wspec, wspec, wspec],
        out_specs=pl.BlockSpec((_RB, _H), lambda i: (i, 0)),
        out_shape=jax.ShapeDtypeStruct((_N, _H), jnp.float32),
    )(agg, r, vec, w0, w1, w2)


def _sc_segsum(tab2, idxp, zrows):
    mesh = plsc.VectorSubcoreMesh(core_axis_name="c", subcore_axis_name="s")

    @functools.partial(
        pl.kernel,
        out_type=jax.ShapeDtypeStruct((2 * _NPAD, _HH), jnp.float32),
        mesh=mesh,
        scratch_types=[
            [pltpu.VMEM((2, 2, _CH), jnp.int32) for _ in range(_NB)],
            [pltpu.VMEM((_CH, _HH), jnp.float32) for _ in range(_NB)],
            pltpu.VMEM_SHARED((_NPAD, _HH), jnp.float32),
            pltpu.VMEM_SHARED((_NPAD, _HH), jnp.float32),
            [pltpu.SemaphoreType.DMA for _ in range(_NB)],
            [pltpu.SemaphoreType.DMA for _ in range(_NB)],
            [pltpu.SemaphoreType.DMA for _ in range(_NB)],
            pltpu.SemaphoreType.DMA,
        ],
        compiler_params=pltpu.CompilerParams(use_tc_tiling_on_sc=False),
    )
    def run(tab_hbm, idx_hbm, z_hbm, out_hbm,
            idxb, rows, tabsp, acc, gsem, ssem, isem, zsem):
        cid = jax.lax.axis_index("c")
        sid = jax.lax.axis_index("s")
        wid = cid * _NS + sid
        row0 = sid * _RPT
        pltpu.async_copy(z_hbm, acc.at[pl.ds(row0, _RPT)], zsem)
        pltpu.sync_copy(tab_hbm.at[pl.ds(cid * _NPAD + row0, _RPT)],
                        tabsp.at[pl.ds(row0, _RPT)])
        pltpu.make_async_copy(z_hbm, acc.at[pl.ds(row0, _RPT)], zsem).wait()
        plsc.subcore_barrier()

        def idx_load(k, par, b):
            pltpu.async_copy(idx_hbm.at[sid * _NCH + k], idxb[b].at[par],
                             isem[b])

        def idx_wait(k, par, b):
            pltpu.make_async_copy(idx_hbm.at[sid * _NCH + k], idxb[b].at[par],
                                  isem[b]).wait()

        for b in range(_NB):
            idx_load(b, 0, b)
        for b in range(_NB):
            idx_wait(b, 0, b)
            pltpu.async_copy(tabsp.at[idxb[b].at[0, 0]], rows[b], gsem[b])
            idx_load(b + _NB, 1, b)

        def round_(g, carry):
            par = jax.lax.rem(g, 2)
            nxt = 1 - par
            for b in range(_NB):
                k = g * _NB + b
                pltpu.make_async_copy(tabsp.at[idxb[b].at[par, 0]],
                                      rows[b], gsem[b]).wait()
                pltpu.async_copy(rows[b], acc.at[idxb[b].at[par, 1]],
                                 ssem[b], add=True)
                pltpu.make_async_copy(rows[b], acc.at[idxb[b].at[par, 1]],
                                      ssem[b]).wait()

                @pl.when(g < _NG - 1)
                def _():
                    idx_wait(k + _NB, nxt, b)
                    pltpu.async_copy(tabsp.at[idxb[b].at[nxt, 0]],
                                     rows[b], gsem[b])

                    @pl.when(g < _NG - 2)
                    def _():
                        idx_load(k + 2 * _NB, par, b)
            return carry

        jax.lax.fori_loop(0, _NG, round_, 0)
        plsc.subcore_barrier()
        pltpu.sync_copy(acc.at[pl.ds(row0, _RPT)],
                        out_hbm.at[pl.ds(cid * _NPAD + row0, _RPT)])

    return run(tab2, idxp, zrows)


def _segsum_full(y, idxp, zrows, zfill):
    tab2 = jnp.concatenate([y[:, :_HH], zfill, y[:, _HH:], zfill], axis=0)
    parts = _sc_segsum(tab2, idxp, zrows)
    return jnp.concatenate([parts[:_N], parts[_NPAD:_NPAD + _N]], axis=1)


def kernel(x, edge_index, Wl0, bl0, Wr0, Wl1, bl1, Wr1,
           g0, be0, g1, be1, LW0, Lb0, LW1, Lb1, LW2, Lb2):
    src = edge_index[0]
    dst = edge_index[1]
    pad = _EPAD - _E
    srcp = jnp.concatenate([src, jnp.zeros((pad,), jnp.int32)])
    dstp = jnp.concatenate([dst, jnp.full((pad,), _N, jnp.int32)])
    idxp = jnp.concatenate([srcp.reshape(_NS * _NCH, 1, _CH),
                            dstp.reshape(_NS * _NCH, 1, _CH)], axis=1)
    zrows = jnp.zeros((_RPT, _HH), jnp.float32)
    zfill = jnp.zeros((_NPAD - _N, _HH), jnp.float32)

    c = 1.0 / np.sqrt(1.0 + _BN_EPS)
    zero = jnp.zeros((_H,), jnp.float32)
    vec_b = jnp.stack([bl0, g0 * c, be0, zero, zero, zero, zero, zero])
    vec_c = jnp.stack([bl1, g1 * c, be1, Lb0, Lb1, Lb2, zero, zero])

    y0, r0 = _stage_a(x, Wl0, Wr0)
    agg0 = _segsum_full(y0, idxp, zrows, zfill)
    y1, r1 = _stage_b(agg0, r0, vec_b, Wl1, Wr1)
    agg1 = _segsum_full(y1, idxp, zrows, zfill)
    return _stage_c(agg1, r1, vec_c, LW0, LW1, LW2)


# speedup vs baseline: 2.3103x; 1.0635x over previous
import functools

import jax
import jax.numpy as jnp
import numpy as np
from jax.experimental import pallas as pl
from jax.experimental.pallas import tpu as pltpu
from jax.experimental.pallas import tpu_sc as plsc

_N, _E, _D, _H = 10000, 320000, 128, 128
_HH = _H // 2
_BN_EPS = 1e-5

_NC, _NS = 2, 16
_NW = _NC * _NS
_CH = 128
_NB = 4
_EPT = 20480
_EPAD = _NS * _EPT
_NCH = _EPT // _CH
_NG = _NCH // _NB
_NPAD = 10112
_RPT = _NPAD // _NS

_RB = 1264
_NBLK = _NPAD // _RB


def _dotT(a, w):
    return jax.lax.dot_general(a, w, (((1,), (1,)), ((), ())),
                               preferred_element_type=jnp.float32)


_TABS = jax.ShapeDtypeStruct((2 * _NPAD, _HH), jnp.float32)


def _half_blk():
    return pl.BlockSpec((_RB, _HH), lambda j, i: (j * _NBLK + i, 0))


def _lo_blk():
    return pl.BlockSpec((_RB, _HH), lambda j, i: (i, 0))


def _hi_blk():
    return pl.BlockSpec((_RB, _HH), lambda j, i: (_NBLK + i, 0))


def _proj_body(x_ref, w_ref, o_ref):
    o_ref[...] = _dotT(x_ref[...], w_ref[...])


def _proj(x, w):
    return pl.pallas_call(
        _proj_body,
        grid=(2, _NBLK),
        in_specs=[pl.BlockSpec((_RB, _D), lambda j, i: (i, 0)),
                  pl.BlockSpec((_HH, _D), lambda j, i: (j, 0))],
        out_specs=_half_blk(),
        out_shape=_TABS,
    )(x, w)


def _mid_body(pl_ref, ph_ref, rl_ref, rh_ref, v_ref, w_ref, o_ref):
    agg = jnp.concatenate([pl_ref[...], ph_ref[...]], axis=1)
    r = jnp.concatenate([rl_ref[...], rh_ref[...]], axis=1)
    z = agg + r + v_ref[0:1, :]
    h = jnp.maximum(z * v_ref[1:2, :] + v_ref[2:3, :], 0.0)
    o_ref[...] = _dotT(h, w_ref[...])


def _mid(parts, rs, vec, w):
    vspec = pl.BlockSpec((8, _H), lambda j, i: (0, 0))
    return pl.pallas_call(
        _mid_body,
        grid=(2, _NBLK),
        in_specs=[_lo_blk(), _hi_blk(), _lo_blk(), _hi_blk(), vspec,
                  pl.BlockSpec((_HH, _H), lambda j, i: (j, 0))],
        out_specs=_half_blk(),
        out_shape=_TABS,
    )(parts, parts, rs, rs, vec, w)


def _head_body(pl_ref, ph_ref, rl_ref, rh_ref, v_ref,
               w0_ref, w1_ref, w2_ref, o_ref):
    agg = jnp.concatenate([pl_ref[...], ph_ref[...]], axis=1)
    r = jnp.concatenate([rl_ref[...], rh_ref[...]], axis=1)
    z = agg + r + v_ref[0:1, :]
    h = jnp.maximum(z * v_ref[1:2, :] + v_ref[2:3, :], 0.0)
    t = jnp.maximum(_dotT(h, w0_ref[...]) + v_ref[3:4, :], 0.0)
    t = jnp.maximum(_dotT(t, w1_ref[...]) + v_ref[4:5, :], 0.0)
    o_ref[...] = _dotT(t, w2_ref[...]) + v_ref[5:6, :]


def _head(parts, rs, vec, w0, w1, w2):
    lo = pl.BlockSpec((_RB, _HH), lambda i: (i, 0))
    hi = pl.BlockSpec((_RB, _HH), lambda i: (_NBLK + i, 0))
    vspec = pl.BlockSpec((8, _H), lambda i: (0, 0))
    wspec = pl.BlockSpec((_H, _H), lambda i: (0, 0))
    return pl.pallas_call(
        _head_body,
        grid=(_NBLK,),
        in_specs=[lo, hi, lo, hi, vspec, wspec, wspec, wspec],
        out_specs=pl.BlockSpec((_RB, _H), lambda i: (i, 0)),
        out_shape=jax.ShapeDtypeStruct((_N, _H), jnp.float32),
    )(parts, parts, rs, rs, vec, w0, w1, w2)


def _sc_segsum(tab, idxp, zrows):
    mesh = plsc.VectorSubcoreMesh(core_axis_name="c", subcore_axis_name="s")

    @functools.partial(
        pl.kernel,
        out_type=_TABS,
        mesh=mesh,
        scratch_types=[
            [pltpu.VMEM((2, 2, _CH), jnp.int32) for _ in range(_NB)],
            [pltpu.VMEM((_CH, _HH), jnp.float32) for _ in range(_NB)],
            pltpu.VMEM_SHARED((_NPAD, _HH), jnp.float32),
            pltpu.VMEM_SHARED((_NPAD, _HH), jnp.float32),
            [pltpu.SemaphoreType.DMA for _ in range(_NB)],
            [pltpu.SemaphoreType.DMA for _ in range(_NB)],
            [pltpu.SemaphoreType.DMA for _ in range(_NB)],
            pltpu.SemaphoreType.DMA,
        ],
        compiler_params=pltpu.CompilerParams(use_tc_tiling_on_sc=False),
    )
    def run(tab_hbm, idx_hbm, z_hbm, out_hbm,
            idxb, rows, tabsp, acc, gsem, ssem, isem, zsem):
        cid = jax.lax.axis_index("c")
        sid = jax.lax.axis_index("s")
        row0 = sid * _RPT
        pltpu.async_copy(z_hbm, acc.at[pl.ds(row0, _RPT)], zsem)
        pltpu.sync_copy(tab_hbm.at[pl.ds(cid * _NPAD + row0, _RPT)],
                        tabsp.at[pl.ds(row0, _RPT)])
        pltpu.make_async_copy(z_hbm, acc.at[pl.ds(row0, _RPT)], zsem).wait()
        plsc.subcore_barrier()

        def idx_load(k, par, b):
            pltpu.async_copy(idx_hbm.at[sid * _NCH + k], idxb[b].at[par],
                             isem[b])

        def idx_wait(k, par, b):
            pltpu.make_async_copy(idx_hbm.at[sid * _NCH + k], idxb[b].at[par],
                                  isem[b]).wait()

        for b in range(_NB):
            idx_load(b, 0, b)
        for b in range(_NB):
            idx_wait(b, 0, b)
            pltpu.async_copy(tabsp.at[idxb[b].at[0, 0]], rows[b], gsem[b])
            idx_load(b + _NB, 1, b)

        def round_(g, carry):
            par = jax.lax.rem(g, 2)
            nxt = 1 - par
            for b in range(_NB):
                k = g * _NB + b
                pltpu.make_async_copy(tabsp.at[idxb[b].at[par, 0]],
                                      rows[b], gsem[b]).wait()
                pltpu.async_copy(rows[b], acc.at[idxb[b].at[par, 1]],
                                 ssem[b], add=True)
                pltpu.make_async_copy(rows[b], acc.at[idxb[b].at[par, 1]],
                                      ssem[b]).wait()

                @pl.when(g < _NG - 1)
                def _():
                    idx_wait(k + _NB, nxt, b)
                    pltpu.async_copy(tabsp.at[idxb[b].at[nxt, 0]],
                                     rows[b], gsem[b])

                    @pl.when(g < _NG - 2)
                    def _():
                        idx_load(k + 2 * _NB, par, b)
            return carry

        jax.lax.fori_loop(0, _NG, round_, 0)
        plsc.subcore_barrier()
        pltpu.sync_copy(acc.at[pl.ds(row0, _RPT)],
                        out_hbm.at[pl.ds(cid * _NPAD + row0, _RPT)])

    return run(tab, idxp, zrows)


def kernel(x, edge_index, Wl0, bl0, Wr0, Wl1, bl1, Wr1,
           g0, be0, g1, be1, LW0, Lb0, LW1, Lb1, LW2, Lb2):
    src = edge_index[0]
    dst = edge_index[1]
    pad = _EPAD - _E
    srcp = jnp.concatenate([src, jnp.zeros((pad,), jnp.int32)])
    dstp = jnp.concatenate([dst, jnp.full((pad,), _N, jnp.int32)])
    idxp = jnp.concatenate([srcp.reshape(_NS * _NCH, 1, _CH),
                            dstp.reshape(_NS * _NCH, 1, _CH)], axis=1)
    zrows = jnp.zeros((_RPT, _HH), jnp.float32)

    c = 1.0 / np.sqrt(1.0 + _BN_EPS)
    zero = jnp.zeros((_H,), jnp.float32)
    vec_b = jnp.stack([bl0, g0 * c, be0, zero, zero, zero, zero, zero])
    vec_c = jnp.stack([bl1, g1 * c, be1, Lb0, Lb1, Lb2, zero, zero])

    y0 = _proj(x, Wl0)
    r0 = _proj(x, Wr0)
    parts0 = _sc_segsum(y0, idxp, zrows)
    y1 = _mid(parts0, r0, vec_b, Wl1)
    r1 = _mid(parts0, r0, vec_b, Wr1)
    parts1 = _sc_segsum(y1, idxp, zrows)
    return _head(parts1, r1, vec_c, LW0, LW1, LW2)
